# baseline (speedup 1.0000x reference)
import math

import jax
import jax.numpy as jnp
from jax import lax
from jax.experimental import pallas as pl
from jax.experimental.pallas import tpu as pltpu
from jax.experimental.pallas import tpu_sc as plsc

D = 1024
N_ROWS = 4 * 8192
NC, NS, L = 2, 16, 16
NW = NC * NS
ROWS_PER_W = N_ROWS // NW
CHUNK = 32
N_CHUNKS = ROWS_PER_W // CHUNK
NVEC = D // L
INV_SQRT_D = 1.0 / math.sqrt(D)


def _sc_body(x_hbm, lab_hbm, emb_hbm, out_hbm, emb_v, labc_v, xbuf):
    wid = lax.axis_index("s") * NC + lax.axis_index("c")
    base = wid * ROWS_PER_W

    pltpu.sync_copy(emb_hbm, emb_v)
    for k in range(3):
        for c in range(NVEC):
            sl = pl.ds(c * L, L)
            emb_v[k, sl] = emb_v[k, sl] * INV_SQRT_D

    def chunk_step(i, _):
        row0 = base + i * CHUNK
        pltpu.sync_copy(lab_hbm.at[pl.ds(row0, CHUNK)], labc_v)
        pltpu.sync_copy(x_hbm.at[pl.ds(row0, CHUNK)], xbuf)

        def group_step(g, _):
            labv = labc_v[pl.ds(g * L, L)]
            for j in range(L):
                l = labv[j]
                r = g * L + j
                for c in range(NVEC):
                    sl = pl.ds(c * L, L)
                    plsc.addupdate(xbuf.at[r, sl], emb_v[l, sl])
            return 0

        lax.fori_loop(0, CHUNK // L, group_step, 0)
        pltpu.sync_copy(xbuf, out_hbm.at[pl.ds(row0, CHUNK)])
        return 0

    lax.fori_loop(0, N_CHUNKS, chunk_step, 0)


def kernel(x, role_labels, emb):
    b, s, d = x.shape
    x2 = x.reshape(b * s, d)
    lab = role_labels.astype(jnp.int32).reshape(b * s)

    mesh = plsc.VectorSubcoreMesh(core_axis_name="c", subcore_axis_name="s")
    sc_call = pl.kernel(
        _sc_body, mesh=mesh,
        out_type=jax.ShapeDtypeStruct((b * s, d), jnp.float32),
        scratch_types=[
            pltpu.VMEM((3, D), jnp.float32),
            pltpu.VMEM((CHUNK,), jnp.int32),
            pltpu.VMEM((CHUNK, D), jnp.float32),
        ],
    )
    out = sc_call(x2, lab, emb)
    return out.reshape(b, s, d)


# speedup vs baseline: 4.0945x; 4.0945x over previous
import math

import jax
import jax.numpy as jnp
from jax import lax
from jax.experimental import pallas as pl
from jax.experimental.pallas import tpu as pltpu
from jax.experimental.pallas import tpu_sc as plsc

D = 1024
N_ROWS = 4 * 8192
INV_SQRT_D = 1.0 / math.sqrt(D)

NC, NS, L = 2, 16, 16
NW = NC * NS
S_SC = 4096
SC_ROWS_PER_W = S_SC // NW
CHUNK = 32
SC_N_CHUNKS = SC_ROWS_PER_W // CHUNK
NVEC = D // L

ROWS_PER_BLOCK = 2048
TC_BLOCK0 = S_SC // ROWS_PER_BLOCK
TC_N_BLOCKS = (N_ROWS - S_SC) // ROWS_PER_BLOCK


def _sc_body(x_hbm, lab_hbm, emb_hbm, out_hbm, emb_v, labc_v, xbuf):
    wid = lax.axis_index("s") * NC + lax.axis_index("c")
    base = wid * SC_ROWS_PER_W

    pltpu.sync_copy(emb_hbm, emb_v)
    for k in range(3):
        for c in range(NVEC):
            sl = pl.ds(c * L, L)
            emb_v[k, sl] = emb_v[k, sl] * INV_SQRT_D

    def chunk_step(i, _):
        row0 = base + i * CHUNK
        pltpu.sync_copy(lab_hbm.at[pl.ds(row0, CHUNK)], labc_v)
        pltpu.sync_copy(x_hbm.at[pl.ds(row0, CHUNK)], xbuf)

        def group_step(g, _):
            labv = labc_v[pl.ds(g * L, L)]
            for j in range(L):
                l = labv[j]
                r = g * L + j
                for c in range(NVEC):
                    sl = pl.ds(c * L, L)
                    plsc.addupdate(xbuf.at[r, sl], emb_v[l, sl])
            return 0

        lax.fori_loop(0, CHUNK // L, group_step, 0)
        pltpu.sync_copy(xbuf, out_hbm.at[pl.ds(row0, CHUNK)])
        return 0

    lax.fori_loop(0, SC_N_CHUNKS, chunk_step, 0)


def _tc_body(lab_ref, x_ref, emb_ref, o_ref):
    lab = lab_ref[0]
    r = lab.shape[-1]
    ohT = (jax.lax.broadcasted_iota(jnp.int32, (3, r), 0) == lab).astype(jnp.float32)
    rows = jax.lax.dot_general(
        ohT, emb_ref[...],
        dimension_numbers=(((0,), (0,)), ((), ())),
        preferred_element_type=jnp.float32,
    )
    o_ref[...] = x_ref[...] + rows * INV_SQRT_D


def kernel(x, role_labels, emb):
    b, s, d = x.shape
    n_rows = b * s
    x2 = x.reshape(n_rows, d)
    lab = role_labels.astype(jnp.int32).reshape(n_rows)

    mesh = plsc.VectorSubcoreMesh(core_axis_name="c", subcore_axis_name="s")
    sc_call = pl.kernel(
        _sc_body, mesh=mesh,
        out_type=jax.ShapeDtypeStruct((S_SC, d), jnp.float32),
        scratch_types=[
            pltpu.VMEM((3, D), jnp.float32),
            pltpu.VMEM((CHUNK,), jnp.int32),
            pltpu.VMEM((CHUNK, D), jnp.float32),
        ],
    )
    sc_out = sc_call(x2, lab, emb)

    g = n_rows // ROWS_PER_BLOCK
    lab3 = lab.reshape(g, 1, ROWS_PER_BLOCK)
    tc_out = pl.pallas_call(
        _tc_body,
        grid=(TC_N_BLOCKS,),
        in_specs=[
            pl.BlockSpec((1, 1, ROWS_PER_BLOCK), lambda i: (i + TC_BLOCK0, 0, 0)),
            pl.BlockSpec((ROWS_PER_BLOCK, d), lambda i: (i + TC_BLOCK0, 0)),
            pl.BlockSpec((3, d), lambda i: (0, 0)),
        ],
        out_specs=pl.BlockSpec((ROWS_PER_BLOCK, d), lambda i: (i + TC_BLOCK0, 0)),
        out_shape=jax.ShapeDtypeStruct((n_rows, d), jnp.float32),
    )(lab3, x2, emb)

    out = lax.dynamic_update_slice(tc_out, sc_out, (0, 0))
    return out.reshape(b, s, d)
